# baseline (speedup 1.0000x reference)
import functools

import jax
import jax.numpy as jnp
from jax import lax
from jax.experimental import pallas as pl
from jax.experimental.pallas import tpu as pltpu
from jax.experimental.pallas import tpu_sc as plsc

_VOCAB = 1000
_VPAD = 1024
_D = 10
_B = 16384
_NC, _NS, _L = 2, 16, 16
_NW = _NC * _NS
_BPW = _B // _NW
_GROUPS = _BPW // _L
_TFLAT = _D * _VPAD


def _tables_body(emb_ref, w1_ref, b1_ref, w2_ref, b2_ref,
                 ptabt_ref, etabt_ref):
    pt = lax.dot_general(w1_ref[...], emb_ref[...], (((1,), (1,)), ((), ())),
                         preferred_element_type=jnp.float32) + b1_ref[...][:, None]
    tt = jnp.tanh(pt)
    st = lax.dot_general(w2_ref[...], tt, (((1,), (0,)), ((), ())),
                         preferred_element_type=jnp.float32) + b2_ref[...][:, None]
    pad = jnp.zeros((_D, _VPAD - _VOCAB), jnp.float32)
    ptabt_ref[...] = jnp.concatenate([pt, pad], axis=1)
    etabt_ref[...] = jnp.concatenate([tt + st, pad], axis=1)


_tables = pl.pallas_call(
    _tables_body,
    out_shape=[jax.ShapeDtypeStruct((_D, _VPAD), jnp.float32),
               jax.ShapeDtypeStruct((_D, _VPAD), jnp.float32)],
)


def _rsqrt_nr(x):
    i = plsc.bitcast(x, jnp.int32)
    i = jnp.int32(0x5F3759DF) - lax.shift_right_logical(i, 1)
    y = plsc.bitcast(i, jnp.float32)
    for _ in range(3):
        y = y * (1.5 - 0.5 * x * y * y)
    return y


_sc_mesh = plsc.VectorSubcoreMesh(core_axis_name="c", subcore_axis_name="s")


@functools.partial(
    pl.kernel,
    mesh=_sc_mesh,
    compiler_params=pltpu.CompilerParams(needs_layout_passes=False),
    out_type=[jax.ShapeDtypeStruct((_B * _D,), jnp.float32),
              jax.ShapeDtypeStruct((_B * _D,), jnp.float32),
              jax.ShapeDtypeStruct((_B,), jnp.float32)],
    scratch_types=[
        pltpu.VMEM((_BPW,), jnp.int32),
        pltpu.VMEM((_BPW,), jnp.int32),
        pltpu.VMEM((_BPW,), jnp.int32),
        pltpu.VMEM((_BPW,), jnp.int32),
        pltpu.VMEM((_TFLAT,), jnp.float32),
        pltpu.VMEM((_TFLAT,), jnp.float32),
        pltpu.VMEM((_BPW * _D,), jnp.float32),
        pltpu.VMEM((_BPW * _D,), jnp.float32),
        pltpu.VMEM((_BPW,), jnp.float32),
        pltpu.VMEM((_BPW,), jnp.float32),
        pltpu.VMEM((_L,), jnp.float32),
        pltpu.VMEM((_NS * _L,), jnp.float32),
        pltpu.VMEM((2 * _L,), jnp.float32),
        pltpu.VMEM_SHARED((_NS * _L,), jnp.float32),
    ],
)
def _sc_main(ptabt_hbm, etabt_hbm, idx1_hbm, idx2_hbm, p_hbm,
             x1_hbm, emb_hbm, sims_hbm,
             idx1_v, idx2_v, fidx1_v, fidx2_v, ptabt_v, etabt_v,
             out1_v, out2_v, cos_v, sims_v, acc_v, parts_v, p_v, parts_sh):
    cid = lax.axis_index("c")
    sid = lax.axis_index("s")
    wid = sid * _NC + cid
    base = wid * _BPW
    fbase = (sid * _NC + (1 - cid)) * _BPW

    pltpu.sync_copy(idx1_hbm.at[pl.ds(base, _BPW)], idx1_v)
    pltpu.sync_copy(idx2_hbm.at[pl.ds(base, _BPW)], idx2_v)
    pltpu.sync_copy(idx1_hbm.at[pl.ds(fbase, _BPW)], fidx1_v)
    pltpu.sync_copy(idx2_hbm.at[pl.ds(fbase, _BPW)], fidx2_v)
    pltpu.sync_copy(ptabt_hbm, ptabt_v)
    pltpu.sync_copy(etabt_hbm, etabt_v)
    pltpu.sync_copy(p_hbm, p_v)

    def group(g, dist_acc):
        o = g * _L
        i1v = idx1_v[pl.ds(o, _L)]
        i2v = idx2_v[pl.ds(o, _L)]
        rowbase = o * _D + lax.iota(jnp.int32, _L) * _D
        dotv = jnp.zeros((_L,), jnp.float32)
        n1v = jnp.zeros((_L,), jnp.float32)
        n2v = jnp.zeros((_L,), jnp.float32)
        for j in range(_D):
            r1 = plsc.load_gather(ptabt_v, [i1v + jnp.int32(j * _VPAD)])
            r2 = plsc.load_gather(etabt_v, [i2v + jnp.int32(j * _VPAD)])
            plsc.store_scatter(out1_v, [rowbase + jnp.int32(j)], r1)
            plsc.store_scatter(out2_v, [rowbase + jnp.int32(j)], r2)
            dotv = dotv + r1 * r2
            n1v = n1v + r1 * r1
            n2v = n2v + r2 * r2
        q = jnp.maximum(n1v * n2v, jnp.float32(1e-16))
        cos_v[pl.ds(o, _L)] = dotv * _rsqrt_nr(q)
        return dist_acc + (n1v + n2v - 2.0 * dotv)

    def fgroup(g, dist_acc):
        o = g * _L
        i1v = fidx1_v[pl.ds(o, _L)]
        i2v = fidx2_v[pl.ds(o, _L)]
        dotv = jnp.zeros((_L,), jnp.float32)
        n1v = jnp.zeros((_L,), jnp.float32)
        n2v = jnp.zeros((_L,), jnp.float32)
        for j in range(_D):
            r1 = plsc.load_gather(ptabt_v, [i1v + jnp.int32(j * _VPAD)])
            r2 = plsc.load_gather(etabt_v, [i2v + jnp.int32(j * _VPAD)])
            dotv = dotv + r1 * r2
            n1v = n1v + r1 * r1
            n2v = n2v + r2 * r2
        return dist_acc + (n1v + n2v - 2.0 * dotv)

    dist_vec = lax.fori_loop(0, _GROUPS, group,
                             jnp.zeros((_L,), jnp.float32))
    dist_vec = lax.fori_loop(0, _GROUPS, fgroup, dist_vec)
    acc_v[...] = dist_vec

    pltpu.sync_copy(acc_v, parts_sh.at[pl.ds(sid * _L, _L)])
    plsc.subcore_barrier()
    pltpu.sync_copy(parts_sh, parts_v)
    total = jnp.zeros((_L,), jnp.float32)
    for w in range(_NS):
        total = total + parts_v[pl.ds(w * _L, _L)]
    dist_sqv = jnp.broadcast_to(jnp.sum(total), (_L,))
    distv = dist_sqv * _rsqrt_nr(jnp.maximum(dist_sqv, jnp.float32(1e-30)))
    p0v = p_v[pl.ds(0, _L)]
    p1v = p_v[pl.ds(_L, _L)]
    addend = p1v * distv

    def axpy(g, _):
        o = g * _L
        sims_v[pl.ds(o, _L)] = p0v * cos_v[pl.ds(o, _L)] + addend
        return 0

    lax.fori_loop(0, _GROUPS, axpy, 0)

    pltpu.sync_copy(out1_v, x1_hbm.at[pl.ds(base * _D, _BPW * _D)])
    pltpu.sync_copy(out2_v, emb_hbm.at[pl.ds(base * _D, _BPW * _D)])
    pltpu.sync_copy(sims_v, sims_hbm.at[pl.ds(base, _BPW)])


def kernel(DPTD_name_1, DPTD_name_2, emb_table, W1, b1, W2, b2, p):
    idx1 = DPTD_name_1.astype(jnp.int32)
    idx2 = DPTD_name_2.astype(jnp.int32)
    ptabt, etabt = _tables(emb_table, W1, b1, W2, b2)
    p_lanes = jnp.concatenate([jnp.broadcast_to(p[0], (_L,)),
                               jnp.broadcast_to(p[1], (_L,))])
    x1f, embf, sims = _sc_main(
        ptabt.reshape(_TFLAT), etabt.reshape(_TFLAT), idx1, idx2, p_lanes)
    return (sims, x1f.reshape(_B, _D), embf.reshape(_B, _D))
